# Initial kernel scaffold; baseline (speedup 1.0000x reference)
import jax
import jax.numpy as jnp
from jax.experimental import pallas as pl


def kernel(x):
    raise NotImplementedError("write your pallas kernel here")


# speedup vs baseline: 1.0546x; 1.0546x over previous
import functools

import jax
import jax.numpy as jnp
from jax import lax
from jax.experimental import pallas as pl
from jax.experimental.pallas import tpu as pltpu
from jax.experimental.pallas import tpu_sc as plsc

B = 4
N = 8193
NP = 8192
D = 768
K = 5734
NOUT = K + 1
ROWS_TOTAL = B * NOUT

_SBLK = 512
_NBLK = 17


def _score_body(x_ref, s_ref):
    xb = x_ref[0]
    ss = jnp.sum(xb * xb, axis=-1)
    s_ref[0, 0] = jnp.sqrt(ss)


def _scores(x):
    out = pl.pallas_call(
        _score_body,
        grid=(B, _NBLK),
        in_specs=[pl.BlockSpec((1, _SBLK, D), lambda b, n: (b, n, 0))],
        out_specs=pl.BlockSpec((1, 1, _SBLK), lambda b, n: (b * _NBLK + n, 0, 0)),
        out_shape=jax.ShapeDtypeStruct((B * _NBLK, 1, _SBLK), jnp.float32),
    )(x)
    return out.reshape(B, _NBLK * _SBLK)[:, :N]


_R, _C = 64, 128


def _row_swap(x, m):
    x5 = x.reshape(B, _R // (2 * m), 2, m, _C)
    sw = jnp.concatenate([x5[:, :, 1:2], x5[:, :, 0:1]], axis=2)
    return sw.reshape(B, _R, _C)


def _lane_swap(x, m):
    up = pltpu.roll(x, _C - m, axis=2)
    dn = pltpu.roll(x, m, axis=2)
    cc = lax.broadcasted_iota(jnp.int32, (B, _R, _C), 2)
    return jnp.where((cc & m) == 0, up, dn)


def _sort_body(s_ref, idx_ref):
    k = s_ref[...]
    rr = lax.broadcasted_iota(jnp.int32, (B, _R, _C), 1)
    cc = lax.broadcasted_iota(jnp.int32, (B, _R, _C), 2)
    pos = cc * _R + rr
    idx = pos
    for p in range(13):
        for q in range(p, -1, -1):
            j = 1 << q
            desc = ((pos >> (p + 1)) & 1) == 0
            lo = (pos & j) == 0
            if j < _R:
                kp = _row_swap(k, j)
                ip = _row_swap(idx, j)
            else:
                m = j // _R
                kp = _lane_swap(k, m)
                ip = _lane_swap(idx, m)
            first = (k > kp) | ((k == kp) & (idx < ip))
            keep = (lo == first) == desc
            k = jnp.where(keep, k, kp)
            idx = jnp.where(keep, idx, ip)
    idx_ref[...] = idx + 1


def _sorted_indices(scores_patch):
    s3 = scores_patch.reshape(B, _C, _R).transpose(0, 2, 1)
    out = pl.pallas_call(
        _sort_body,
        out_shape=jax.ShapeDtypeStruct((B, _R, _C), jnp.int32),
    )(s3)
    return out.transpose(0, 2, 1).reshape(B, NP)


_NW = 32
_RPW = 720
_CH = 120
_NCH = _RPW // _CH
_PAD = _NW * _RPW - ROWS_TOTAL

@functools.cache
def _gather_rows_kernel():
    mesh = plsc.VectorSubcoreMesh(core_axis_name="c", subcore_axis_name="s")

    @functools.partial(
        pl.kernel,
        out_type=jax.ShapeDtypeStruct((ROWS_TOTAL, D), jnp.float32),
        mesh=mesh,
        scratch_types=[
            pltpu.VMEM((_NCH, _CH), jnp.int32),
            pltpu.VMEM((_NCH, _CH), jnp.int32),
            pltpu.VMEM((_CH, D), jnp.float32),
            pltpu.SemaphoreType.DMA,
            pltpu.SemaphoreType.DMA,
        ],
    )
    def _gather_rows(x2_hbm, g_hbm, d_hbm, out_hbm, gidx_v, didx_v, rows_v,
                     gsem, wsem):
        wid = lax.axis_index("s") * 2 + lax.axis_index("c")
        pltpu.sync_copy(g_hbm.at[wid], gidx_v)
        pltpu.sync_copy(d_hbm.at[wid], didx_v)
        for ch in range(_NCH):
            pltpu.async_copy(x2_hbm.at[gidx_v.at[ch]], rows_v, gsem).wait()
            pltpu.async_copy(rows_v, out_hbm.at[didx_v.at[ch]], wsem).wait()

    return _gather_rows


def kernel(x):
    scores = _scores(x)[:, 1:]
    srt = _sorted_indices(scores)
    keep = jnp.concatenate(
        [jnp.zeros((B, 1), jnp.int32), srt[:, :K]], axis=1)
    flat_g = (keep + jnp.arange(B, dtype=jnp.int32)[:, None] * N).reshape(-1)
    g = jnp.concatenate([flat_g, flat_g[ROWS_TOTAL - _PAD:]])
    dst = jnp.arange(ROWS_TOTAL, dtype=jnp.int32)
    dvec = jnp.concatenate([dst, dst[ROWS_TOTAL - _PAD:]])
    out = _gather_rows_kernel()(
        x.reshape(B * N, D),
        g.reshape(_NW, _NCH, _CH),
        dvec.reshape(_NW, _NCH, _CH),
    )
    return out.reshape(B, NOUT, D)
